# baseline (speedup 1.0000x reference)
import jax
import jax.numpy as jnp
from jax import lax
from jax.experimental import pallas as pl
from jax.experimental.pallas import tpu as pltpu

_BM = 400
_DNT = (((1,), (1,)), ((), ()))


def _main_body(adj_ref, seq1_ref, seq2_ref, w_ref, b_ref, msk_ref,
               wb_ref, sb1_ref, sb2_ref, a_ref, bb_ref, s_ref, fts_ref, h_scr):
    i = pl.program_id(0)
    ng = pl.num_programs(0)
    nh = w_ref.shape[0]

    @pl.when(i == 0)
    def _init_fts():
        w16 = w_ref[...].astype(jnp.bfloat16)
        fts_ref[:, :nh] = lax.dot_general(
            seq1_ref[...].astype(jnp.bfloat16), w16, _DNT,
            preferred_element_type=jnp.float32).astype(jnp.bfloat16)
        fts_ref[:, nh:] = lax.dot_general(
            seq2_ref[...].astype(jnp.bfloat16), w16, _DNT,
            preferred_element_type=jnp.float32).astype(jnp.bfloat16)

    b = b_ref[...]
    b2 = jnp.concatenate([b, b], axis=1)
    h = jnp.dot(adj_ref[...].astype(jnp.bfloat16), fts_ref[...],
                preferred_element_type=jnp.float32)
    h = h + b2
    h = jnp.where(h >= 0.0, h, a_ref[0] * h)
    h_scr[pl.ds(i * _BM, _BM), :] = h.astype(jnp.bfloat16)

    @pl.when(i == ng - 1)
    def _score():
        msk = msk_ref[...]
        msk16 = msk.astype(jnp.bfloat16)
        inv = 1.0 / jnp.sum(msk)
        hsum = jnp.dot(msk16, h_scr[:, :nh],
                       preferred_element_type=jnp.float32)
        c = jax.nn.sigmoid(hsum * inv)
        v = lax.dot_general(c, wb_ref[...], _DNT,
                            preferred_element_type=jnp.float32)
        z = jnp.zeros_like(v)
        row = lax.broadcasted_iota(jnp.int32, (8, 2 * nh), 0)
        v1 = jnp.broadcast_to(jnp.concatenate([v, z], axis=1), (8, 2 * nh))
        v2 = jnp.broadcast_to(jnp.concatenate([z, v], axis=1), (8, 2 * nh))
        vp = jnp.where(row == 0, v1, 0.0) + jnp.where(row == 1, v2, 0.0)
        s8 = lax.dot_general(vp.astype(jnp.bfloat16), h_scr[...], _DNT,
                             preferred_element_type=jnp.float32)
        bb0 = bb_ref[0]
        s_ref[0:1, :] = s8[0:1, :] + sb1_ref[...] + bb0
        s_ref[1:2, :] = s8[1:2, :] + sb2_ref[...] + bb0


def kernel(seq1, seq2, adj, sparse, msk, samp_bias1, samp_bias2, W, b, a, Wb, bb):
    n = seq1.shape[1]
    nh = W.shape[0]
    adj2 = adj.reshape(n, n)
    s1 = seq1.reshape(n, -1)
    s2 = seq2.reshape(n, -1)

    grid = n // _BM
    S = pl.pallas_call(
        _main_body,
        grid=(grid,),
        in_specs=[
            pl.BlockSpec((_BM, n), lambda i: (i, 0)),
            pl.BlockSpec((n, nh), lambda i: (0, 0)),
            pl.BlockSpec((n, nh), lambda i: (0, 0)),
            pl.BlockSpec((nh, nh), lambda i: (0, 0)),
            pl.BlockSpec((1, nh), lambda i: (0, 0)),
            pl.BlockSpec((1, n), lambda i: (0, 0)),
            pl.BlockSpec((nh, nh), lambda i: (0, 0)),
            pl.BlockSpec((1, n), lambda i: (0, 0)),
            pl.BlockSpec((1, n), lambda i: (0, 0)),
            pl.BlockSpec(memory_space=pltpu.SMEM),
            pl.BlockSpec(memory_space=pltpu.SMEM),
        ],
        out_specs=pl.BlockSpec((2, n), lambda i: (0, 0)),
        out_shape=jax.ShapeDtypeStruct((2, n), jnp.float32),
        scratch_shapes=[
            pltpu.VMEM((n, 2 * nh), jnp.bfloat16),
            pltpu.VMEM((n, 2 * nh), jnp.bfloat16),
        ],
        compiler_params=pltpu.CompilerParams(
            dimension_semantics=("arbitrary",),
            vmem_limit_bytes=100 * 1024 * 1024),
    )(adj2, s1, s2, W, b.reshape(1, nh), msk, Wb.reshape(nh, nh),
      samp_bias1, samp_bias2, a, bb)

    return S.reshape(1, 2 * n)


# speedup vs baseline: 1.0179x; 1.0179x over previous
import jax
import jax.numpy as jnp
from jax import lax
from jax.experimental import pallas as pl
from jax.experimental.pallas import tpu as pltpu

_BM = 400
_DNT = (((1,), (1,)), ((), ()))


def _main_body(adj_ref, seq1_ref, seq2_ref, w_ref, b_ref, msk_ref,
               wb_ref, sb1_ref, sb2_ref, a_ref, bb_ref, s_ref, fts_ref, h_scr):
    i = pl.program_id(0)
    ng = pl.num_programs(0)
    nh = w_ref.shape[0]

    @pl.when(i == 0)
    def _init_fts():
        w0 = w_ref[...]
        fts_ref[:, :nh] = lax.dot_general(
            seq1_ref[...], w0, _DNT,
            preferred_element_type=jnp.float32)
        fts_ref[:, nh:] = lax.dot_general(
            seq2_ref[...], w0, _DNT,
            preferred_element_type=jnp.float32)

    b = b_ref[...]
    b2 = jnp.concatenate([b, b], axis=1)
    h = jnp.dot(adj_ref[...], fts_ref[...],
                preferred_element_type=jnp.float32)
    h = h + b2
    h = jnp.where(h >= 0.0, h, a_ref[0] * h)
    h_scr[pl.ds(i * _BM, _BM), :] = h.astype(jnp.bfloat16)

    @pl.when(i == ng - 1)
    def _score():
        msk = msk_ref[...]
        msk16 = msk.astype(jnp.bfloat16)
        inv = 1.0 / jnp.sum(msk)
        hsum = jnp.dot(msk16, h_scr[:, :nh],
                       preferred_element_type=jnp.float32)
        c = jax.nn.sigmoid(hsum * inv)
        v = lax.dot_general(c, wb_ref[...], _DNT,
                            preferred_element_type=jnp.float32)
        z = jnp.zeros_like(v)
        row = lax.broadcasted_iota(jnp.int32, (8, 2 * nh), 0)
        v1 = jnp.broadcast_to(jnp.concatenate([v, z], axis=1), (8, 2 * nh))
        v2 = jnp.broadcast_to(jnp.concatenate([z, v], axis=1), (8, 2 * nh))
        vp = jnp.where(row == 0, v1, 0.0) + jnp.where(row == 1, v2, 0.0)
        s8 = lax.dot_general(vp.astype(jnp.bfloat16), h_scr[...], _DNT,
                             preferred_element_type=jnp.float32)
        bb0 = bb_ref[0]
        s_ref[0:1, :] = s8[0:1, :] + sb1_ref[...] + bb0
        s_ref[1:2, :] = s8[1:2, :] + sb2_ref[...] + bb0


def kernel(seq1, seq2, adj, sparse, msk, samp_bias1, samp_bias2, W, b, a, Wb, bb):
    n = seq1.shape[1]
    nh = W.shape[0]
    adj2 = adj.reshape(n, n)
    s1 = seq1.reshape(n, -1)
    s2 = seq2.reshape(n, -1)

    grid = n // _BM
    S = pl.pallas_call(
        _main_body,
        grid=(grid,),
        in_specs=[
            pl.BlockSpec((_BM, n), lambda i: (i, 0)),
            pl.BlockSpec((n, nh), lambda i: (0, 0)),
            pl.BlockSpec((n, nh), lambda i: (0, 0)),
            pl.BlockSpec((nh, nh), lambda i: (0, 0)),
            pl.BlockSpec((1, nh), lambda i: (0, 0)),
            pl.BlockSpec((1, n), lambda i: (0, 0)),
            pl.BlockSpec((nh, nh), lambda i: (0, 0)),
            pl.BlockSpec((1, n), lambda i: (0, 0)),
            pl.BlockSpec((1, n), lambda i: (0, 0)),
            pl.BlockSpec(memory_space=pltpu.SMEM),
            pl.BlockSpec(memory_space=pltpu.SMEM),
        ],
        out_specs=pl.BlockSpec((2, n), lambda i: (0, 0)),
        out_shape=jax.ShapeDtypeStruct((2, n), jnp.float32),
        scratch_shapes=[
            pltpu.VMEM((n, 2 * nh), jnp.float32),
            pltpu.VMEM((n, 2 * nh), jnp.bfloat16),
        ],
        compiler_params=pltpu.CompilerParams(
            dimension_semantics=("arbitrary",),
            vmem_limit_bytes=100 * 1024 * 1024),
    )(adj2, s1, s2, W, b.reshape(1, nh), msk, Wb.reshape(nh, nh),
      samp_bias1, samp_bias2, a, bb)

    return S.reshape(1, 2 * n)
